# Initial kernel scaffold; baseline (speedup 1.0000x reference)
import jax
import jax.numpy as jnp
from jax.experimental import pallas as pl


def kernel(im1, im2, C, M1, M2):
    raise NotImplementedError("write your pallas kernel here")


# speedup vs baseline: 20.1916x; 20.1916x over previous
import functools

import jax
import jax.numpy as jnp
from jax import lax
from jax.experimental import pallas as pl
from jax.experimental.pallas import tpu as pltpu
from jax.experimental.pallas import tpu_sc as plsc

D = 224
HW = D * D
N = 32
NC = 2
NS = 16
ROWS_PER_CHUNK = 8
CHUNK = ROWS_PER_CHUNK * D
NUM_CHUNKS = D // ROWS_PER_CHUNK
VPR = D // 16

_LO = 0.001
_HI = D - 1.001
_DF = float(D)


def _floor_ceil(p):
    fi = p.astype(jnp.int32)
    ff = fi.astype(jnp.float32)
    exact = p == ff
    ci = jnp.where(exact, fi, fi + 1)
    cf = ci.astype(jnp.float32)
    return fi, ff, ci, cf


def _branch(im_v, pxr, pyr):
    px = jnp.minimum(jnp.maximum(pxr, _LO), _HI)
    py = jnp.minimum(jnp.maximum(pyr, _LO), _HI)
    ifx, fxf, icx, cxf = _floor_ceil(px)
    ify, fyf, icy, cyf = _floor_ceil(py)
    wfx = 1.0 - (px - fxf)
    wcx = 1.0 - (cxf - px)
    wfy = 1.0 - (py - fyf)
    wcy = 1.0 - (cyf - py)
    rf = ifx * D
    rc = icx * D
    res = (wfx * wfy) * plsc.load_gather(im_v, [rf + ify])
    res = res + (wcx * wfy) * plsc.load_gather(im_v, [rc + ify])
    res = res + (wfx * wcy) * plsc.load_gather(im_v, [rf + icy])
    res = res + (wcx * wcy) * plsc.load_gather(im_v, [rc + icy])
    dx = pxr - px
    dy = pyr - py
    return res, dx * dx + dy * dy


def _body(im1_hbm, im2_hbm, c_hbm, m1_hbm, m2_hbm, out_hbm, loss_hbm,
          im1_v, im2_v, c0_v, c1_v, m1_v, m2_v, out_v, loss_v):
    wid = lax.axis_index("s") * NC + lax.axis_index("c")
    n = wid
    loss_v[:] = jnp.zeros((16,), jnp.float32)
    iota = lax.iota(jnp.int32, 16)

    def channel_body(ch, _):
        base_im = (n * 3 + ch) * HW
        pltpu.sync_copy(im1_hbm.at[pl.ds(base_im, HW)], im1_v)
        pltpu.sync_copy(im2_hbm.at[pl.ds(base_im, HW)], im2_v)

        def chunk_body(cidx, _):
            off = cidx * CHUNK
            pltpu.sync_copy(c_hbm.at[pl.ds(n * 2 * HW + off, CHUNK)], c0_v)
            pltpu.sync_copy(c_hbm.at[pl.ds(n * 2 * HW + HW + off, CHUNK)], c1_v)
            pltpu.sync_copy(m1_hbm.at[pl.ds(n * HW + off, CHUNK)], m1_v)
            pltpu.sync_copy(m2_hbm.at[pl.ds(n * HW + off, CHUNK)], m2_v)

            def row_body(r8, _):
                rowf = (cidx * ROWS_PER_CHUNK + r8).astype(jnp.float32)
                acc = jnp.zeros((16,), jnp.float32)
                for j in range(VPR):
                    s = r8 * D + j * 16
                    colf = (iota + j * 16).astype(jnp.float32)
                    c0 = c0_v[pl.ds(s, 16)]
                    c1 = c1_v[pl.ds(s, 16)]
                    r1, l1 = _branch(im1_v, (rowf + c0) * _DF,
                                     (colf + c1) * _DF)
                    r2, l2 = _branch(im2_v, (rowf - c0) * _DF,
                                     (colf - c1) * _DF)
                    m1 = m1_v[pl.ds(s, 16)]
                    m2 = m2_v[pl.ds(s, 16)]
                    out_v[pl.ds(s, 16)] = (m1 * r1 + m2 * r2) / (m1 + m2)
                    acc = acc + l1 + l2
                loss_v[:] = loss_v[:] + acc
                return 0

            lax.fori_loop(0, ROWS_PER_CHUNK, row_body, 0)
            pltpu.sync_copy(out_v, out_hbm.at[pl.ds(base_im + off, CHUNK)])
            return 0

        lax.fori_loop(0, NUM_CHUNKS, chunk_body, 0)
        return 0

    lax.fori_loop(0, 3, channel_body, 0)
    pltpu.sync_copy(loss_v, loss_hbm.at[pl.ds(wid * 16, 16)])


@jax.jit
def _run(im1f, im2f, cf, m1f, m2f):
    mesh = plsc.VectorSubcoreMesh(core_axis_name="c", subcore_axis_name="s")
    k = pl.kernel(
        _body,
        out_type=[
            jax.ShapeDtypeStruct((N * 3 * HW,), jnp.float32),
            jax.ShapeDtypeStruct((NC * NS * 16,), jnp.float32),
        ],
        mesh=mesh,
        compiler_params=pltpu.CompilerParams(needs_layout_passes=False),
        scratch_types=[
            pltpu.VMEM((HW,), jnp.float32),
            pltpu.VMEM((HW,), jnp.float32),
            pltpu.VMEM((CHUNK,), jnp.float32),
            pltpu.VMEM((CHUNK,), jnp.float32),
            pltpu.VMEM((CHUNK,), jnp.float32),
            pltpu.VMEM((CHUNK,), jnp.float32),
            pltpu.VMEM((CHUNK,), jnp.float32),
            pltpu.VMEM((16,), jnp.float32),
        ],
    )
    return k(im1f, im2f, cf, m1f, m2f)


def kernel(im1, im2, C, M1, M2):
    out_flat, loss_part = _run(
        im1.reshape(N * 3 * HW),
        im2.reshape(N * 3 * HW),
        C.reshape(N * 2 * HW),
        M1.reshape(N * HW),
        M2.reshape(N * HW),
    )
    out = out_flat.reshape(N, 3, D, D)
    scale = 1e-4 / (3.0 * N * 2 * HW * D * D)
    loss = loss_part.sum() * jnp.float32(scale)
    return out, loss


# speedup vs baseline: 30.2715x; 1.4992x over previous
import jax
import jax.numpy as jnp
from jax import lax
from jax.experimental import pallas as pl
from jax.experimental.pallas import tpu as pltpu
from jax.experimental.pallas import tpu_sc as plsc

D = 224
HW = D * D
N = 32
NC = 2
NS = 16
ROWS_PER_CHUNK = 8
CHUNK = ROWS_PER_CHUNK * D
NUM_CHUNKS = D // ROWS_PER_CHUNK
VPR = D // 16

_LO = 0.001
_HI = D - 1.001
_DF = float(D)


def _floor_ceil(p):
    fi = p.astype(jnp.int32)
    ff = fi.astype(jnp.float32)
    exact = p == ff
    ci = jnp.where(exact, fi, fi + 1)
    cf = ci.astype(jnp.float32)
    return fi, ff, ci, cf


def _branch(im_v, pxr, pyr):
    px = jnp.minimum(jnp.maximum(pxr, _LO), _HI)
    py = jnp.minimum(jnp.maximum(pyr, _LO), _HI)
    ifx, fxf, icx, cxf = _floor_ceil(px)
    ify, fyf, icy, cyf = _floor_ceil(py)
    wfx = 1.0 - (px - fxf)
    wcx = 1.0 - (cxf - px)
    wfy = 1.0 - (py - fyf)
    wcy = 1.0 - (cyf - py)
    rf = ifx * D
    rc = icx * D
    res = (wfx * wfy) * plsc.load_gather(im_v, [rf + ify])
    res = res + (wcx * wfy) * plsc.load_gather(im_v, [rc + ify])
    res = res + (wfx * wcy) * plsc.load_gather(im_v, [rf + icy])
    res = res + (wcx * wcy) * plsc.load_gather(im_v, [rc + icy])
    dx = pxr - px
    dy = pyr - py
    return res, dx * dx + dy * dy


def _body(im1_hbm, im2_hbm, c_hbm, m1_hbm, m2_hbm, out_hbm, loss_hbm,
          im1_v, im2_v,
          c0_a, c1_a, m1_a, m2_a, c0_b, c1_b, m1_b, m2_b,
          out_a, out_b, loss_v,
          in_sem_a, in_sem_b, out_sem_a, out_sem_b):
    wid = lax.axis_index("s") * NC + lax.axis_index("c")
    n = wid
    n2hw = n * 2 * HW
    nhw = n * HW
    loss_v[:] = jnp.zeros((16,), jnp.float32)
    iota = lax.iota(jnp.int32, 16)

    bufs_a = (c0_a, c1_a, m1_a, m2_a)
    bufs_b = (c0_b, c1_b, m1_b, m2_b)

    def in_srcs(c):
        off = c * CHUNK
        return (c_hbm.at[pl.ds(n2hw + off, CHUNK)],
                c_hbm.at[pl.ds(n2hw + HW + off, CHUNK)],
                m1_hbm.at[pl.ds(nhw + off, CHUNK)],
                m2_hbm.at[pl.ds(nhw + off, CHUNK)])

    def start_in(c, bufs, sem):
        for src, dst in zip(in_srcs(c), bufs):
            pltpu.async_copy(src, dst, sem)

    def wait_in(c, bufs, sem):
        for src, dst in zip(in_srcs(c), bufs):
            pltpu.make_async_copy(src, dst, sem).wait()

    def start_out(base_im, c, buf, sem):
        pltpu.async_copy(buf, out_hbm.at[pl.ds(base_im + c * CHUNK, CHUNK)],
                         sem)

    def wait_out(base_im, c, buf, sem):
        pltpu.make_async_copy(
            buf, out_hbm.at[pl.ds(base_im + c * CHUNK, CHUNK)], sem).wait()

    def compute_chunk(cidx, bufs, out_v):
        c0_v, c1_v, m1_v, m2_v = bufs

        def row_body(r8, _):
            rowf = (cidx * ROWS_PER_CHUNK + r8).astype(jnp.float32)
            acc = jnp.zeros((16,), jnp.float32)
            for j in range(VPR):
                s = r8 * D + j * 16
                colf = (iota + j * 16).astype(jnp.float32)
                c0 = c0_v[pl.ds(s, 16)]
                c1 = c1_v[pl.ds(s, 16)]
                r1, l1 = _branch(im1_v, (rowf + c0) * _DF, (colf + c1) * _DF)
                r2, l2 = _branch(im2_v, (rowf - c0) * _DF, (colf - c1) * _DF)
                m1 = m1_v[pl.ds(s, 16)]
                m2 = m2_v[pl.ds(s, 16)]
                out_v[pl.ds(s, 16)] = (m1 * r1 + m2 * r2) / (m1 + m2)
                acc = acc + l1 + l2
            loss_v[:] = loss_v[:] + acc
            return 0

        lax.fori_loop(0, ROWS_PER_CHUNK, row_body, 0)

    def channel_body(ch, _):
        base_im = (n * 3 + ch) * HW
        start_in(0, bufs_a, in_sem_a)
        pltpu.sync_copy(im1_hbm.at[pl.ds(base_im, HW)], im1_v)
        pltpu.sync_copy(im2_hbm.at[pl.ds(base_im, HW)], im2_v)

        def pair_body(p, _):
            ca = 2 * p
            cb = 2 * p + 1
            start_in(cb, bufs_b, in_sem_b)
            wait_in(ca, bufs_a, in_sem_a)

            @pl.when(p > 0)
            def _():
                wait_out(base_im, ca - 2, out_a, out_sem_a)

            compute_chunk(ca, bufs_a, out_a)
            start_out(base_im, ca, out_a, out_sem_a)

            @pl.when(p < NUM_CHUNKS // 2 - 1)
            def _():
                start_in(ca + 2, bufs_a, in_sem_a)

            wait_in(cb, bufs_b, in_sem_b)

            @pl.when(p > 0)
            def _():
                wait_out(base_im, cb - 2, out_b, out_sem_b)

            compute_chunk(cb, bufs_b, out_b)
            start_out(base_im, cb, out_b, out_sem_b)
            return 0

        lax.fori_loop(0, NUM_CHUNKS // 2, pair_body, 0)
        wait_out(base_im, NUM_CHUNKS - 2, out_a, out_sem_a)
        wait_out(base_im, NUM_CHUNKS - 1, out_b, out_sem_b)
        return 0

    lax.fori_loop(0, 3, channel_body, 0)
    pltpu.sync_copy(loss_v, loss_hbm.at[pl.ds(wid * 16, 16)])


@jax.jit
def _run(im1f, im2f, cf, m1f, m2f):
    mesh = plsc.VectorSubcoreMesh(core_axis_name="c", subcore_axis_name="s")
    chunk_f32 = pltpu.VMEM((CHUNK,), jnp.float32)
    k = pl.kernel(
        _body,
        out_type=[
            jax.ShapeDtypeStruct((N * 3 * HW,), jnp.float32),
            jax.ShapeDtypeStruct((NC * NS * 16,), jnp.float32),
        ],
        mesh=mesh,
        compiler_params=pltpu.CompilerParams(needs_layout_passes=False),
        scratch_types=[
            pltpu.VMEM((HW,), jnp.float32),
            pltpu.VMEM((HW,), jnp.float32),
            chunk_f32, chunk_f32, chunk_f32, chunk_f32,
            chunk_f32, chunk_f32, chunk_f32, chunk_f32,
            chunk_f32, chunk_f32,
            pltpu.VMEM((16,), jnp.float32),
            pltpu.SemaphoreType.DMA,
            pltpu.SemaphoreType.DMA,
            pltpu.SemaphoreType.DMA,
            pltpu.SemaphoreType.DMA,
        ],
    )
    return k(im1f, im2f, cf, m1f, m2f)


def kernel(im1, im2, C, M1, M2):
    out_flat, loss_part = _run(
        im1.reshape(N * 3 * HW),
        im2.reshape(N * 3 * HW),
        C.reshape(N * 2 * HW),
        M1.reshape(N * HW),
        M2.reshape(N * HW),
    )
    out = out_flat.reshape(N, 3, D, D)
    scale = 1e-4 / (3.0 * N * 2 * HW * D * D)
    loss = loss_part.sum() * jnp.float32(scale)
    return out, loss
